# baseline (speedup 1.0000x reference)
import functools

import jax
import jax.numpy as jnp
from jax import lax
from jax.experimental import pallas as pl
from jax.experimental.pallas import tpu as pltpu
from jax.experimental.pallas import tpu_sc as plsc

N = 10000
E = 320000
F = 128

NC = 2
NS = 16
NW = NC * NS
CHUNK = 16
EPS = 20480
E_PAD = NS * EPS
CPW = EPS // CHUNK
G = 32
NGRP = CPW // G
GROWS = 2 * G * CHUNK // 128

SLAB = 10112
HALFN = 5000
ACC = 5120
TRASH = ACC - HALFN
ACC_PER_TILE = ACC // NS
SLAB_PER_TILE = SLAB // NS

_mesh = plsc.VectorSubcoreMesh(core_axis_name="c", subcore_axis_name="s")


@functools.partial(
    pl.kernel,
    out_type=jax.ShapeDtypeStruct((NC, NS, ACC_PER_TILE, F), jnp.float32),
    mesh=_mesh,
    scratch_types=[
        [pltpu.VMEM((GROWS, 128), jnp.int32) for _ in range(2)],
        [pltpu.VMEM((CHUNK, F), jnp.float32) for _ in range(2)],
        pltpu.VMEM_SHARED((ACC, F), jnp.float32),
        pltpu.VMEM_SHARED((SLAB, F), jnp.float32),
        [pltpu.SemaphoreType.DMA for _ in range(2)],
        [pltpu.SemaphoreType.DMA for _ in range(2)],
        [pltpu.SemaphoreType.DMA for _ in range(2)],
    ],
)
def _segsum_sc(idx_hbm, xs_hbm, out_hbm, idxs, bufs, acc, xs,
               gsems, ssems, isems):
    c = lax.axis_index("c")
    s = lax.axis_index("s")
    wid = s * NC + c

    def cvec(iv, jj):
        return iv[jj // 4, pl.ds((jj % 4) * 32, 16)]

    def rvec(iv, jj):
        return iv[jj // 4, pl.ds((jj % 4) * 32 + 16, 16)]

    z = jnp.zeros((16,), jnp.float32)

    def zrow(i, carry):
        for j in range(F // 16):
            bufs[0][i, pl.ds(j * 16, 16)] = z
            bufs[1][i, pl.ds(j * 16, 16)] = z
        return carry

    lax.fori_loop(0, CHUNK, zrow, 0)
    acc_base = s * ACC_PER_TILE
    nz = ACC_PER_TILE // CHUNK
    for k in range(nz):
        pltpu.async_copy(
            bufs[k % 2], acc.at[pl.ds(acc_base + k * CHUNK, CHUNK)],
            gsems[k % 2],
        )
    sl = s * SLAB_PER_TILE
    pltpu.sync_copy(
        xs_hbm.at[pl.ds(sl, SLAB_PER_TILE)], xs.at[pl.ds(sl, SLAB_PER_TILE)]
    )
    for k in range(nz):
        pltpu.make_async_copy(
            bufs[k % 2], acc.at[pl.ds(acc_base + k * CHUNK, CHUNK)],
            gsems[k % 2],
        ).wait()
    plsc.subcore_barrier()

    pltpu.sync_copy(idx_hbm.at[wid, 0], idxs[0])
    pltpu.async_copy(xs.at[cvec(idxs[0], 0)], bufs[0], gsems[0])
    pltpu.async_copy(bufs[1], acc.at[rvec(idxs[0], 0)], ssems[1], add=True)

    def outer(g2, carry):
        for par in range(2):
            h = g2 * 2 + par
            iv = idxs[par]
            ivn = idxs[1 - par]
            for jj in range(G):
                b = jj % 2
                pltpu.make_async_copy(
                    xs.at[cvec(iv, jj)], bufs[b], gsems[b]
                ).wait()
                if jj == 2:
                    pltpu.async_copy(
                        idx_hbm.at[wid, h + 1], ivn, isems[1 - par]
                    )
                pltpu.async_copy(
                    bufs[b], acc.at[rvec(iv, jj)], ssems[b], add=True
                )
                pltpu.make_async_copy(
                    bufs[1 - b], acc.at[rvec(iv, jj)], ssems[1 - b]
                ).wait()
                if jj < G - 1:
                    pltpu.async_copy(
                        xs.at[cvec(iv, jj + 1)], bufs[1 - b], gsems[1 - b]
                    )
                else:
                    pltpu.make_async_copy(
                        idx_hbm.at[wid, h + 1], ivn, isems[1 - par]
                    ).wait()
                    pltpu.async_copy(
                        xs.at[cvec(ivn, 0)], bufs[1 - b], gsems[1 - b]
                    )
        return carry

    lax.fori_loop(0, NGRP // 2, outer, 0)

    pltpu.make_async_copy(
        xs.at[cvec(idxs[0], 0)], bufs[0], gsems[0]
    ).wait()
    pltpu.make_async_copy(
        bufs[1], acc.at[rvec(idxs[0], 0)], ssems[1]
    ).wait()

    plsc.subcore_barrier()
    pltpu.sync_copy(
        acc.at[pl.ds(s * ACC_PER_TILE, ACC_PER_TILE)], out_hbm.at[c, s]
    )


ROWS_BLK = 1000
_GRID = N // ROWS_BLK


def _combine_body(p_ref, w_ref, o_ref):
    y = jnp.dot(p_ref[0], w_ref[...], preferred_element_type=jnp.float32)
    o_ref[...] = jnp.where(y > 0, y, jnp.exp(y) - 1.0)


def _combine(parts, W):
    return pl.pallas_call(
        _combine_body,
        grid=(_GRID,),
        in_specs=[
            pl.BlockSpec((1, ROWS_BLK, F), lambda i: (i // 5, i % 5, 0)),
            pl.BlockSpec((F, F), lambda i: (0, 0)),
        ],
        out_specs=pl.BlockSpec((ROWS_BLK, F), lambda i: (i, 0)),
        out_shape=jax.ShapeDtypeStruct((N, F), jnp.float32),
    )(parts, W)


def kernel(input, edge_index, W, a):
    row = edge_index[0]
    col = edge_index[1]
    pad = E_PAD - E
    col_p = jnp.concatenate([col, jnp.zeros((pad,), jnp.int32)])
    row_p = jnp.concatenate([row, jnp.full((pad,), -1, jnp.int32)])
    colr = col_p.reshape(NS, EPS)
    rowr = row_p.reshape(NS, EPS)
    e2 = jnp.arange(EPS, dtype=jnp.int32)
    spread = (e2 % SLAB)[None, :]
    trash_l = (HALFN + e2 % TRASH)[None, :]
    half = rowr // HALFN
    phased = []
    for c in range(NC):
        ih = half == c
        cl = jnp.where(ih, colr, spread)
        rl = jnp.where(ih, rowr - c * HALFN, trash_l)
        phased.append(jnp.stack(
            [cl.reshape(NS, CPW, CHUNK), rl.reshape(NS, CPW, CHUNK)], axis=3
        ))
    idx = jnp.stack(phased, axis=1)
    idx = jnp.swapaxes(idx, 3, 4)
    idx = idx.reshape(NW, NGRP, GROWS, 128)
    idx = jnp.concatenate(
        [idx, jnp.zeros((NW, 1, GROWS, 128), jnp.int32)], axis=1
    )

    x_pad = jnp.concatenate([input, jnp.zeros((SLAB - N, F), jnp.float32)])

    parts = _segsum_sc(idx, x_pad)
    parts = parts.reshape(NC, ACC, F)
    return _combine(parts, W)


# speedup vs baseline: 1.3855x; 1.3855x over previous
import functools

import jax
import jax.numpy as jnp
from jax import lax
from jax.experimental import pallas as pl
from jax.experimental.pallas import tpu as pltpu
from jax.experimental.pallas import tpu_sc as plsc

N = 10000
E = 320000
F = 128

NC = 2
NS = 16
NW = NC * NS
CHUNK = 32
EPS = 20480
E_PAD = NS * EPS
CPW = EPS // CHUNK
G = 8
NGRP = CPW // G

SLAB = 10000
HALFN = 5000
ACC = 5120
TRASH = ACC - HALFN
ACC_PER_TILE = ACC // NS
SLAB_PER_TILE = 632

_mesh = plsc.VectorSubcoreMesh(core_axis_name="c", subcore_axis_name="s")


@functools.partial(
    pl.kernel,
    out_type=jax.ShapeDtypeStruct((NC, NS, ACC_PER_TILE, F), jnp.float32),
    mesh=_mesh,
    scratch_types=[
        [pltpu.VMEM((2, 128), jnp.int32) for _ in range(2)],
        [pltpu.VMEM((2, 128), jnp.int32) for _ in range(2)],
        [pltpu.VMEM((CHUNK, F), jnp.float32) for _ in range(2)],
        pltpu.VMEM_SHARED((ACC, F), jnp.float32),
        pltpu.VMEM_SHARED((SLAB, F), jnp.float32),
        [pltpu.SemaphoreType.DMA for _ in range(2)],
        [pltpu.SemaphoreType.DMA for _ in range(2)],
        [pltpu.SemaphoreType.DMA for _ in range(2)],
    ],
)
def _segsum_sc(gidx_hbm, sidx_hbm, xs_hbm, out_hbm, gidxs, sidxs, bufs,
               acc, xs, gsems, ssems, isems):
    c = lax.axis_index("c")
    s = lax.axis_index("s")
    wid = s * NC + c

    def cref(par, jj):
        return gidxs[par].at[jj // 4, pl.ds((jj % 4) * 32, CHUNK)]

    def rref(par, jj):
        return sidxs[par].at[jj // 4, pl.ds((jj % 4) * 32, CHUNK)]

    z = jnp.zeros((16,), jnp.float32)

    def zrow(i, carry):
        for j in range(F // 16):
            bufs[0][i, pl.ds(j * 16, 16)] = z
            bufs[1][i, pl.ds(j * 16, 16)] = z
        return carry

    lax.fori_loop(0, CHUNK, zrow, 0)
    acc_base = s * ACC_PER_TILE
    nz = ACC_PER_TILE // CHUNK
    for k in range(nz):
        pltpu.async_copy(
            bufs[k % 2], acc.at[pl.ds(acc_base + k * CHUNK, CHUNK)],
            gsems[k % 2],
        )
    sl = s * SLAB_PER_TILE

    @pl.when(s < NS - 1)
    def _():
        pltpu.sync_copy(
            xs_hbm.at[pl.ds(sl, SLAB_PER_TILE)],
            xs.at[pl.ds(sl, SLAB_PER_TILE)],
        )

    @pl.when(s == NS - 1)
    def _():
        pltpu.sync_copy(
            xs_hbm.at[pl.ds(sl, SLAB - 15 * SLAB_PER_TILE)],
            xs.at[pl.ds(sl, SLAB - 15 * SLAB_PER_TILE)],
        )
    for k in range(nz):
        pltpu.make_async_copy(
            bufs[k % 2], acc.at[pl.ds(acc_base + k * CHUNK, CHUNK)],
            gsems[k % 2],
        ).wait()
    plsc.subcore_barrier()

    pltpu.sync_copy(gidx_hbm.at[wid, 0], gidxs[0])
    pltpu.sync_copy(sidx_hbm.at[wid, 0], sidxs[0])
    pltpu.async_copy(xs.at[cref(0, 0)], bufs[0], gsems[0])
    pltpu.async_copy(bufs[1], acc.at[rref(0, 0)], ssems[1], add=True)

    def outer(g2, carry):
        for par in range(2):
            h = g2 * 2 + par
            for jj in range(G):
                b = jj % 2
                pltpu.make_async_copy(
                    xs.at[cref(par, jj)], bufs[b], gsems[b]
                ).wait()
                if jj == 2:
                    pltpu.async_copy(
                        gidx_hbm.at[wid, h + 1], gidxs[1 - par],
                        isems[1 - par],
                    )
                    pltpu.async_copy(
                        sidx_hbm.at[wid, h + 1], sidxs[1 - par],
                        isems[1 - par],
                    )
                pltpu.async_copy(
                    bufs[b], acc.at[rref(par, jj)], ssems[b], add=True
                )
                pltpu.make_async_copy(
                    bufs[1 - b], acc.at[rref(par, jj)], ssems[1 - b]
                ).wait()
                if jj < G - 1:
                    pltpu.async_copy(
                        xs.at[cref(par, jj + 1)], bufs[1 - b], gsems[1 - b]
                    )
                else:
                    pltpu.make_async_copy(
                        gidx_hbm.at[wid, h + 1], gidxs[1 - par],
                        isems[1 - par],
                    ).wait()
                    pltpu.make_async_copy(
                        sidx_hbm.at[wid, h + 1], sidxs[1 - par],
                        isems[1 - par],
                    ).wait()
                    pltpu.async_copy(
                        xs.at[cref(1 - par, 0)], bufs[1 - b], gsems[1 - b]
                    )
        return carry

    lax.fori_loop(0, NGRP // 2, outer, 0)

    pltpu.make_async_copy(
        xs.at[cref(0, 0)], bufs[0], gsems[0]
    ).wait()
    pltpu.make_async_copy(
        bufs[1], acc.at[rref(0, 0)], ssems[1]
    ).wait()

    plsc.subcore_barrier()
    pltpu.sync_copy(
        acc.at[pl.ds(s * ACC_PER_TILE, ACC_PER_TILE)], out_hbm.at[c, s]
    )


ROWS_BLK = 1000
_GRID = N // ROWS_BLK


def _combine_body(p_ref, w_ref, o_ref):
    y = jnp.dot(p_ref[0], w_ref[...], preferred_element_type=jnp.float32)
    o_ref[...] = jnp.where(y > 0, y, jnp.exp(y) - 1.0)


def _combine(parts, W):
    return pl.pallas_call(
        _combine_body,
        grid=(_GRID,),
        in_specs=[
            pl.BlockSpec((1, ROWS_BLK, F), lambda i: (i // 5, i % 5, 0)),
            pl.BlockSpec((F, F), lambda i: (0, 0)),
        ],
        out_specs=pl.BlockSpec((ROWS_BLK, F), lambda i: (i, 0)),
        out_shape=jax.ShapeDtypeStruct((N, F), jnp.float32),
    )(parts, W)


def kernel(input, edge_index, W, a):
    row = edge_index[0]
    col = edge_index[1]
    pad = E_PAD - E
    col_p = jnp.concatenate([col, jnp.zeros((pad,), jnp.int32)])
    row_p = jnp.concatenate([row, jnp.full((pad,), -1, jnp.int32)])
    colr = col_p.reshape(NS, EPS)
    rowr = row_p.reshape(NS, EPS)
    e2 = jnp.arange(EPS, dtype=jnp.int32)
    spread = (e2 % SLAB)[None, :]
    trash_l = (HALFN + e2 % TRASH)[None, :]
    half = rowr // HALFN
    phased = []
    for c in range(NC):
        ih = half == c
        cl = jnp.where(ih, colr, spread)
        rl = jnp.where(ih, rowr - c * HALFN, trash_l)
        phased.append(jnp.stack(
            [cl.reshape(NS, CPW, CHUNK), rl.reshape(NS, CPW, CHUNK)], axis=-1
        ))
    gidx = jnp.stack([p[..., 0] for p in phased], axis=1)
    gidx = gidx.reshape(NW, NGRP, 2, 128)
    gidx = jnp.concatenate(
        [gidx, jnp.zeros((NW, 1, 2, 128), jnp.int32)], axis=1
    )
    sidx = jnp.stack([p[..., 1] for p in phased], axis=1)
    sidx = sidx.reshape(NW, NGRP, 2, 128)
    sidx = jnp.concatenate(
        [sidx, jnp.zeros((NW, 1, 2, 128), jnp.int32)], axis=1
    )

    parts = _segsum_sc(gidx, sidx, input)
    parts = parts.reshape(NC, ACC, F)
    return _combine(parts, W)
